# baseline (speedup 1.0000x reference)
import functools

import jax
import jax.numpy as jnp
from jax import lax
from jax.experimental import pallas as pl
from jax.experimental.pallas import tpu as pltpu
from jax.experimental.pallas import tpu_sc as plsc

N = 10000
E = 160000
D_IN = 256
D_HID = 512

NC = 2
NS = 16
L = 16

NP = 10240
EP = 163840
EW = 32
EB = EP // EW
ERT = EB // NS
ERW = EB // (NC * NS)
NDW = NP // NS
DGR = NP // 8
SLAB = 16

_SC_MESH = plsc.VectorSubcoreMesh(
    core_axis_name="c", subcore_axis_name="s", num_cores=NC, num_subcores=NS)
_SC_PARAMS = pltpu.CompilerParams(needs_layout_passes=False)

_IOTA = lambda: lax.iota(jnp.int32, L)


def _rsqrt16(v):
  i = lax.bitcast_convert_type(v, jnp.int32)
  i = jnp.full((L,), 0x5F3759DF, dtype=jnp.int32) - lax.shift_right_logical(i, 1)
  y = lax.bitcast_convert_type(i, jnp.float32)
  for _ in range(3):
    y = y * (1.5 - 0.5 * v * y * y)
  return y


def _zero_rows(ref, nrows, ncol16):
  @pl.loop(0, nrows)
  def _(i):
    for k in range(ncol16):
      ref[i, pl.ds(16 * k, 16)] = jnp.zeros((L,), jnp.float32)


def _phase_a_body(row2, col2, ew2, dinv_out, norm_out,
                  rbuf, cbuf, ebuf, c8buf, stage, dchunk2, dbuf, nbuf,
                  degacc, dshared2):
  c = lax.axis_index("c")
  s = lax.axis_index("s")
  zero16 = jnp.zeros((L,), jnp.float32)

  _zero_rows(stage, EW, 8)
  pltpu.sync_copy(stage, degacc.at[pl.ds(s * 80, 32)])
  pltpu.sync_copy(stage, degacc.at[pl.ds(s * 80 + 32, 32)])
  pltpu.sync_copy(stage.at[pl.ds(0, 16)], degacc.at[pl.ds(s * 80 + 64, 16)])
  plsc.subcore_barrier()

  @pl.loop(0, ERT // 8)
  def _(cc):
    base = s * ERT + cc * 8
    pltpu.sync_copy(col2.at[pl.ds(base, 8)], cbuf)
    pltpu.sync_copy(ew2.at[pl.ds(base, 8)], ebuf)

    @pl.loop(0, 8)
    def _(j):
      for i in range(2):
        sl = pl.ds(16 * i, 16)
        cv16 = cbuf[j, sl]
        c8buf[j, sl] = lax.shift_right_logical(cv16, 3)
        ew16 = ebuf[j, sl]
        for q in range(16):
          grp = jnp.bitwise_and(cv16[q], 7)
          ws = jnp.full((L,), ew16[q], jnp.float32)
          for g2 in range(8):
            stage[16 * i + q, pl.ds(16 * g2, 16)] = jnp.where(
                grp == g2, ws, zero16)
      pltpu.sync_copy(stage, degacc.at[c8buf.at[j]], add=True)

  plsc.subcore_barrier()

  @pl.when(s < 10)
  def _():
    for r in range(8):
      pltpu.sync_copy(degacc.at[pl.ds(128 * s + 16 * r, 16)],
                      stage.at[pl.ds(0, 16)])
      for t in range(8):
        tot = zero16
        for rr in range(2):
          for q in range(8):
            tot = jnp.where(_IOTA() == 8 * rr + q,
                            stage[2 * t + rr, pl.ds(16 * q, 16)], tot)
        dchunk2[r, pl.ds(16 * t, 16)] = _rsqrt16(tot + 1.0)

    pltpu.sync_copy(dchunk2, dshared2.at[pl.ds(8 * s, 8)])

    @pl.when(c == 0)
    def _():
      pltpu.sync_copy(dchunk2, dinv_out.at[pl.ds(8 * s, 8)])

  plsc.subcore_barrier()

  pltpu.sync_copy(dshared2, dbuf)

  w = s * NC + c

  @pl.loop(0, ERW // 8)
  def _(cc):
    base = w * ERW + cc * 8
    pltpu.sync_copy(row2.at[pl.ds(base, 8)], rbuf)
    pltpu.sync_copy(col2.at[pl.ds(base, 8)], cbuf)
    pltpu.sync_copy(ew2.at[pl.ds(base, 8)], ebuf)

    @pl.loop(0, 8)
    def _(j):
      for i in range(2):
        sl = pl.ds(16 * i, 16)
        r16 = rbuf[j, sl]
        c16 = cbuf[j, sl]
        dr = plsc.load_gather(
            dbuf, [lax.shift_right_logical(r16, 7),
                   jnp.bitwise_and(r16, 127)])
        dc = plsc.load_gather(
            dbuf, [lax.shift_right_logical(c16, 7),
                   jnp.bitwise_and(c16, 127)])
        nbuf[j, sl] = dr * ebuf[j, sl] * dc

    pltpu.sync_copy(nbuf, norm_out.at[pl.ds(base, 8)])


_phase_a = functools.partial(
    pl.kernel,
    out_type=(jax.ShapeDtypeStruct((NP // 128, 128), jnp.float32),
              jax.ShapeDtypeStruct((EB, EW), jnp.float32)),
    mesh=_SC_MESH,
    scratch_types=[
        pltpu.VMEM((8, EW), jnp.int32),
        pltpu.VMEM((8, EW), jnp.int32),
        pltpu.VMEM((8, EW), jnp.float32),
        pltpu.VMEM((8, EW), jnp.int32),
        pltpu.VMEM((EW, 128), jnp.float32),
        pltpu.VMEM((8, 128), jnp.float32),
        pltpu.VMEM((NP // 128, 128), jnp.float32),
        pltpu.VMEM((8, EW), jnp.float32),
        pltpu.VMEM_SHARED((DGR, 128), jnp.float32),
        pltpu.VMEM_SHARED((NP // 128, 128), jnp.float32),
    ],
    compiler_params=_SC_PARAMS,
)(_phase_a_body)


def _msg_pass_main(xt, row2, col2, norm2, s, rowbuf, colbuf, normbuf,
                   g0, g1, s0, s1, gsem0, gsem1, ssem0, ssem1,
                   lwbuf, lbbuf, acc):
  gb = (g0, g1)
  sb = (s0, s1)
  gsems = (gsem0, gsem1)
  ssems = (ssem0, ssem1)

  def gstart(j, b):
    pltpu.async_copy(xt.at[rowbuf.at[j]], gb[b], gsems[b])

  def gwait(j, b):
    pltpu.make_async_copy(xt.at[rowbuf.at[j]], gb[b], gsems[b]).wait()

  def sstart(j, b):
    pltpu.async_copy(sb[b], acc.at[colbuf.at[j]], ssems[b], add=True)

  def swait(j, b):
    pltpu.make_async_copy(sb[b], acc.at[colbuf.at[j]], ssems[b]).wait()

  @pl.loop(0, ERT // SLAB)
  def _(cc):
    base = s * ERT + cc * SLAB
    pltpu.sync_copy(row2.at[pl.ds(base, SLAB)], rowbuf)
    pltpu.sync_copy(col2.at[pl.ds(base, SLAB)], colbuf)
    pltpu.sync_copy(norm2.at[pl.ds(base, SLAB)], normbuf)
    gstart(0, 0)
    gstart(1, 1)

    @pl.loop(0, SLAB // 2)
    def _(jj):
      for b in range(2):
        j = jj * 2 + b
        gwait(j, b)

        @pl.when(jnp.logical_or(cc > 0, jj > 0))
        def _():
          swait(jnp.maximum(j - 2, 0), b)

        @pl.loop(0, 2)
        def _(eo):
          nv = normbuf[j, pl.ds(eo * 16, 16)]
          for q in range(16):
            ns = jnp.full((L,), nv[q], jnp.float32)
            e = eo * 16 + q
            for k in range(8):
              sl = pl.ds(16 * k, 16)
              v = (gb[b][e, sl] + ns * lwbuf[pl.ds(16 * k, 16)]
                   + lbbuf[pl.ds(16 * k, 16)])
              sb[b][e, sl] = jnp.maximum(v, 0.0)
        sstart(j, b)

        @pl.when(jj < SLAB // 2 - 1)
        def _():
          gstart(j + 2, b)

  for b in range(2):
    swait(SLAB - 2 + b, b)


def _phase_b_body(x0, x1, row2, col2, norm2, lw2, lb2, out0, out1,
                  rowbuf, colbuf, normbuf, g0, g1, s0, s1,
                  gsem0, gsem1, ssem0, ssem1, lwbuf, lbbuf, acc):
  c = lax.axis_index("c")
  s = lax.axis_index("s")

  pltpu.sync_copy(lw2.at[c], lwbuf)
  pltpu.sync_copy(lb2.at[c], lbbuf)

  _zero_rows(g0, EW, 8)
  for t in range(NDW // EW):
    pltpu.sync_copy(g0, acc.at[pl.ds(s * NDW + EW * t, EW)])
  plsc.subcore_barrier()

  args = (row2, col2, norm2, s, rowbuf, colbuf, normbuf, g0, g1, s0, s1,
          gsem0, gsem1, ssem0, ssem1, lwbuf, lbbuf, acc)

  @pl.when(c == 0)
  def _():
    _msg_pass_main(x0, *args)

  @pl.when(c == 1)
  def _():
    _msg_pass_main(x1, *args)

  plsc.subcore_barrier()
  for t in range(NDW // EW):
    sl = pl.ds(s * NDW + EW * t, EW)
    pltpu.sync_copy(acc.at[sl], g0)

    @pl.when(c == 0)
    def _():
      pltpu.sync_copy(g0, out0.at[sl])

    @pl.when(c == 1)
    def _():
      pltpu.sync_copy(g0, out1.at[sl])


_phase_b = functools.partial(
    pl.kernel,
    out_type=(jax.ShapeDtypeStruct((NP, 128), jnp.float32),
              jax.ShapeDtypeStruct((NP, 128), jnp.float32)),
    mesh=_SC_MESH,
    scratch_types=[
        pltpu.VMEM((SLAB, EW), jnp.int32),
        pltpu.VMEM((SLAB, EW), jnp.int32),
        pltpu.VMEM((SLAB, EW), jnp.float32),
        pltpu.VMEM((EW, 128), jnp.float32),
        pltpu.VMEM((EW, 128), jnp.float32),
        pltpu.VMEM((EW, 128), jnp.float32),
        pltpu.VMEM((EW, 128), jnp.float32),
        pltpu.SemaphoreType.DMA,
        pltpu.SemaphoreType.DMA,
        pltpu.SemaphoreType.DMA,
        pltpu.SemaphoreType.DMA,
        pltpu.VMEM((128,), jnp.float32),
        pltpu.VMEM((128,), jnp.float32),
        pltpu.VMEM_SHARED((NP, 128), jnp.float32),
    ],
    compiler_params=_SC_PARAMS,
)(_phase_b_body)


_ROWS_BLK = 640
_INVS = float((1.0 + 1e-5) ** -0.5)


def _mlp_kernel(eps_ref, dinv_ref, x_ref, a0_ref, a1_ref, lw_ref, lb_ref,
                w1_ref, b1_ref, g1_ref, be1_ref, w2_ref, b2_ref,
                bng_ref, bnb_ref, out_ref, *, final_bn):
  x = x_ref[...]
  aggr = jnp.concatenate([a0_ref[...], a1_ref[...]], axis=1)
  d = dinv_ref[...]
  sl_msg = jnp.maximum(x + (d * d) * lw_ref[...] + lb_ref[...], 0.0)
  hin = (1.0 + eps_ref[0, 0]) * x + aggr + sl_msg
  t = jnp.dot(hin, w1_ref[...], preferred_element_type=jnp.float32) + b1_ref[...]
  t = jnp.maximum(g1_ref[...] * (t * _INVS) + be1_ref[...], 0.0)
  h = jnp.dot(t, w2_ref[...], preferred_element_type=jnp.float32) + b2_ref[...]
  if final_bn:
    h = jnp.maximum(bng_ref[...] * (h * _INVS) + bnb_ref[...], 0.0)
  out_ref[...] = h


def _mlp_call(final_bn, d_out, eps, dinv, x, a0, a1, lw, lb, w1, b1, g1, be1,
              w2, b2, bng, bnb):
  full = lambda shape: pl.BlockSpec(shape, lambda i: (0, 0))
  grid = NP // _ROWS_BLK
  return pl.pallas_call(
      functools.partial(_mlp_kernel, final_bn=final_bn),
      grid=(grid,),
      in_specs=[
          full((1, 1)),
          pl.BlockSpec((_ROWS_BLK, 1), lambda i: (i, 0)),
          pl.BlockSpec((_ROWS_BLK, D_IN), lambda i: (i, 0)),
          pl.BlockSpec((_ROWS_BLK, 128), lambda i: (i, 0)),
          pl.BlockSpec((_ROWS_BLK, 128), lambda i: (i, 0)),
          full((1, D_IN)), full((1, D_IN)),
          full((D_IN, D_HID)), full((1, D_HID)), full((1, D_HID)),
          full((1, D_HID)),
          full((D_HID, d_out)), full((1, d_out)),
          full((1, d_out)), full((1, d_out)),
      ],
      out_specs=pl.BlockSpec((_ROWS_BLK, d_out), lambda i: (i, 0)),
      out_shape=jax.ShapeDtypeStruct((NP, d_out), jnp.float32),
  )(eps.reshape(1, 1), dinv.reshape(NP, 1), x, a0, a1,
    lw.reshape(1, D_IN), lb.reshape(1, D_IN), w1, b1.reshape(1, D_HID),
    g1.reshape(1, D_HID), be1.reshape(1, D_HID), w2, b2.reshape(1, d_out),
    bng.reshape(1, d_out), bnb.reshape(1, d_out))


def kernel(x, edge_index, edge_weight, eps0, le0_w, le0_b, m0_w1, m0_b1,
           m0_g1, m0_be1, m0_w2, m0_b2, bn0_g, bn0_b, eps1, le1_w, le1_b,
           m1_w1, m1_b1, m1_g1, m1_be1, m1_w2, m1_b2):
  pad = EP - E
  row2 = jnp.concatenate(
      [edge_index[0], jnp.zeros((pad,), jnp.int32)]).reshape(EB, EW)
  col2 = jnp.concatenate(
      [edge_index[1], jnp.full((pad,), N, jnp.int32)]).reshape(EB, EW)
  ew2 = jnp.concatenate(
      [edge_weight, jnp.zeros((pad,), jnp.float32)]).reshape(EB, EW)

  dinv2, norm2 = _phase_a(row2, col2, ew2)
  dinv = dinv2.reshape(NP)

  xp = jnp.pad(x, ((0, NP - N), (0, 0)))
  a0, a1 = _phase_b(x[:, :128], x[:, 128:], row2, col2, norm2,
                    le0_w.reshape(2, 128), le0_b.reshape(2, 128))
  h = _mlp_call(True, D_IN, eps0, dinv, xp, a0, a1, le0_w, le0_b,
                m0_w1, m0_b1, m0_g1, m0_be1, m0_w2, m0_b2, bn0_g, bn0_b)

  b0, b1 = _phase_b(h[:N, :128], h[:N, 128:], row2, col2, norm2,
                    le1_w.reshape(2, 128), le1_b.reshape(2, 128))
  out = _mlp_call(False, D_IN, eps1, dinv, h, b0, b1, le1_w, le1_b,
                  m1_w1, m1_b1, m1_g1, m1_be1, m1_w2, m1_b2,
                  jnp.ones((D_IN,), jnp.float32), jnp.zeros((D_IN,), jnp.float32))
  return out[:N]


# speedup vs baseline: 1.8818x; 1.8818x over previous
import functools

import jax
import jax.numpy as jnp
from jax import lax
from jax.experimental import pallas as pl
from jax.experimental.pallas import tpu as pltpu
from jax.experimental.pallas import tpu_sc as plsc

N = 10000
E = 160000
D_IN = 256
D_HID = 512

NC = 2
NS = 16
L = 16

NP = 10240
EP = 163840
EW = 32
EB = EP // EW
ERT = EB // NS
ERW = EB // (NC * NS)
NDW = NP // NS
DGR = NP // 8
SLAB = 32

_SC_MESH = plsc.VectorSubcoreMesh(
    core_axis_name="c", subcore_axis_name="s", num_cores=NC, num_subcores=NS)
_SC_PARAMS = pltpu.CompilerParams(needs_layout_passes=False)

_IOTA = lambda: lax.iota(jnp.int32, L)


def _rsqrt16(v):
  i = lax.bitcast_convert_type(v, jnp.int32)
  i = jnp.full((L,), 0x5F3759DF, dtype=jnp.int32) - lax.shift_right_logical(i, 1)
  y = lax.bitcast_convert_type(i, jnp.float32)
  for _ in range(3):
    y = y * (1.5 - 0.5 * v * y * y)
  return y


def _zero_rows(ref, nrows, ncol16):
  @pl.loop(0, nrows)
  def _(i):
    for k in range(ncol16):
      ref[i, pl.ds(16 * k, 16)] = jnp.zeros((L,), jnp.float32)


def _phase_a_body(row2, col2, ew2, dinv_out, norm_out,
                  rbuf, cbuf, ebuf, c8buf, stage, dchunk2, dbuf, nbuf,
                  degacc, dshared2):
  c = lax.axis_index("c")
  s = lax.axis_index("s")
  zero16 = jnp.zeros((L,), jnp.float32)

  _zero_rows(stage, EW, 8)
  pltpu.sync_copy(stage, degacc.at[pl.ds(s * 80, 32)])
  pltpu.sync_copy(stage, degacc.at[pl.ds(s * 80 + 32, 32)])
  pltpu.sync_copy(stage.at[pl.ds(0, 16)], degacc.at[pl.ds(s * 80 + 64, 16)])
  plsc.subcore_barrier()

  @pl.loop(0, ERT // 8)
  def _(cc):
    base = s * ERT + cc * 8
    pltpu.sync_copy(col2.at[pl.ds(base, 8)], cbuf)
    pltpu.sync_copy(ew2.at[pl.ds(base, 8)], ebuf)

    @pl.loop(0, 8)
    def _(j):
      for i in range(2):
        sl = pl.ds(16 * i, 16)
        cv16 = cbuf[j, sl]
        c8buf[j, sl] = lax.shift_right_logical(cv16, 3)
        ew16 = ebuf[j, sl]
        for q in range(16):
          grp = jnp.bitwise_and(cv16[q], 7)
          ws = jnp.full((L,), ew16[q], jnp.float32)
          for g2 in range(8):
            stage[16 * i + q, pl.ds(16 * g2, 16)] = jnp.where(
                grp == g2, ws, zero16)
      pltpu.sync_copy(stage, degacc.at[c8buf.at[j]], add=True)

  plsc.subcore_barrier()

  @pl.when(s < 10)
  def _():
    for r in range(8):
      pltpu.sync_copy(degacc.at[pl.ds(128 * s + 16 * r, 16)],
                      stage.at[pl.ds(0, 16)])
      for t in range(8):
        tot = zero16
        for rr in range(2):
          for q in range(8):
            tot = jnp.where(_IOTA() == 8 * rr + q,
                            stage[2 * t + rr, pl.ds(16 * q, 16)], tot)
        dchunk2[r, pl.ds(16 * t, 16)] = _rsqrt16(tot + 1.0)

    pltpu.sync_copy(dchunk2, dshared2.at[pl.ds(8 * s, 8)])

    @pl.when(c == 0)
    def _():
      pltpu.sync_copy(dchunk2, dinv_out.at[pl.ds(8 * s, 8)])

  plsc.subcore_barrier()

  pltpu.sync_copy(dshared2, dbuf)

  w = s * NC + c

  @pl.loop(0, ERW // 8)
  def _(cc):
    base = w * ERW + cc * 8
    pltpu.sync_copy(row2.at[pl.ds(base, 8)], rbuf)
    pltpu.sync_copy(col2.at[pl.ds(base, 8)], cbuf)
    pltpu.sync_copy(ew2.at[pl.ds(base, 8)], ebuf)

    @pl.loop(0, 8)
    def _(j):
      for i in range(2):
        sl = pl.ds(16 * i, 16)
        r16 = rbuf[j, sl]
        c16 = cbuf[j, sl]
        dr = plsc.load_gather(
            dbuf, [lax.shift_right_logical(r16, 7),
                   jnp.bitwise_and(r16, 127)])
        dc = plsc.load_gather(
            dbuf, [lax.shift_right_logical(c16, 7),
                   jnp.bitwise_and(c16, 127)])
        nbuf[j, sl] = dr * ebuf[j, sl] * dc

    pltpu.sync_copy(nbuf, norm_out.at[pl.ds(base, 8)])


_phase_a = functools.partial(
    pl.kernel,
    out_type=(jax.ShapeDtypeStruct((NP // 128, 128), jnp.float32),
              jax.ShapeDtypeStruct((EB, EW), jnp.float32)),
    mesh=_SC_MESH,
    scratch_types=[
        pltpu.VMEM((8, EW), jnp.int32),
        pltpu.VMEM((8, EW), jnp.int32),
        pltpu.VMEM((8, EW), jnp.float32),
        pltpu.VMEM((8, EW), jnp.int32),
        pltpu.VMEM((EW, 128), jnp.float32),
        pltpu.VMEM((8, 128), jnp.float32),
        pltpu.VMEM((NP // 128, 128), jnp.float32),
        pltpu.VMEM((8, EW), jnp.float32),
        pltpu.VMEM_SHARED((DGR, 128), jnp.float32),
        pltpu.VMEM_SHARED((NP // 128, 128), jnp.float32),
    ],
    compiler_params=_SC_PARAMS,
)(_phase_a_body)


_RING = 4


def _msg_pass_main(xt, row2, col2, norm2, s, rowbuf, colbuf, normbuf,
                   g0, g1, g2, g3, gsem0, gsem1, gsem2, gsem3,
                   lwbuf, lbbuf, acc):
  gb = (g0, g1, g2, g3)
  gsems = (gsem0, gsem1, gsem2, gsem3)
  lwv = [lwbuf[pl.ds(16 * k, 16)] for k in range(8)]
  lbv = [lbbuf[pl.ds(16 * k, 16)] for k in range(8)]

  def gstart(j, b):
    pltpu.async_copy(xt.at[rowbuf.at[j]], gb[b], gsems[b])

  def gwait(j, b):
    pltpu.make_async_copy(xt.at[rowbuf.at[j]], gb[b], gsems[b]).wait()

  @pl.loop(0, ERT // SLAB)
  def _(cc):
    base = s * ERT + cc * SLAB
    pltpu.sync_copy(row2.at[pl.ds(base, SLAB)], rowbuf)
    pltpu.sync_copy(col2.at[pl.ds(base, SLAB)], colbuf)
    pltpu.sync_copy(norm2.at[pl.ds(base, SLAB)], normbuf)
    for b in range(_RING):
      gstart(b, b)

    @pl.loop(0, SLAB // _RING)
    def _(jj):
      for b in range(_RING):
        j = jj * _RING + b
        gwait(j, b)
        for eo in range(2):
          nv = normbuf[j, pl.ds(eo * 16, 16)]
          for q in range(16):
            ns = jnp.full((L,), nv[q], jnp.float32)
            e = eo * 16 + q
            for k in range(8):
              sl = pl.ds(16 * k, 16)
              v = gb[b][e, sl] + (ns * lwv[k] + lbv[k])
              gb[b][e, sl] = jnp.maximum(v, 0.0)
        pltpu.sync_copy(gb[b], acc.at[colbuf.at[j]], add=True)

        @pl.when(jj < SLAB // _RING - 1)
        def _():
          gstart(j + _RING, b)


def _phase_b_body(x0, x1, row2, col2, norm2, lw2, lb2, out0, out1,
                  rowbuf, colbuf, normbuf, g0, g1, g2, g3,
                  gsem0, gsem1, gsem2, gsem3, lwbuf, lbbuf, acc):
  c = lax.axis_index("c")
  s = lax.axis_index("s")

  pltpu.sync_copy(lw2.at[c], lwbuf)
  pltpu.sync_copy(lb2.at[c], lbbuf)

  _zero_rows(g0, EW, 8)
  for t in range(NDW // EW):
    pltpu.sync_copy(g0, acc.at[pl.ds(s * NDW + EW * t, EW)])
  plsc.subcore_barrier()

  args = (row2, col2, norm2, s, rowbuf, colbuf, normbuf, g0, g1, g2, g3,
          gsem0, gsem1, gsem2, gsem3, lwbuf, lbbuf, acc)

  @pl.when(c == 0)
  def _():
    _msg_pass_main(x0, *args)

  @pl.when(c == 1)
  def _():
    _msg_pass_main(x1, *args)

  plsc.subcore_barrier()
  for t in range(NDW // EW):
    sl = pl.ds(s * NDW + EW * t, EW)
    pltpu.sync_copy(acc.at[sl], g0)

    @pl.when(c == 0)
    def _():
      pltpu.sync_copy(g0, out0.at[sl])

    @pl.when(c == 1)
    def _():
      pltpu.sync_copy(g0, out1.at[sl])


_phase_b = functools.partial(
    pl.kernel,
    out_type=(jax.ShapeDtypeStruct((NP, 128), jnp.float32),
              jax.ShapeDtypeStruct((NP, 128), jnp.float32)),
    mesh=_SC_MESH,
    scratch_types=[
        pltpu.VMEM((SLAB, EW), jnp.int32),
        pltpu.VMEM((SLAB, EW), jnp.int32),
        pltpu.VMEM((SLAB, EW), jnp.float32),
        pltpu.VMEM((EW, 128), jnp.float32),
        pltpu.VMEM((EW, 128), jnp.float32),
        pltpu.VMEM((EW, 128), jnp.float32),
        pltpu.VMEM((EW, 128), jnp.float32),
        pltpu.SemaphoreType.DMA,
        pltpu.SemaphoreType.DMA,
        pltpu.SemaphoreType.DMA,
        pltpu.SemaphoreType.DMA,
        pltpu.VMEM((128,), jnp.float32),
        pltpu.VMEM((128,), jnp.float32),
        pltpu.VMEM_SHARED((NP, 128), jnp.float32),
    ],
    compiler_params=_SC_PARAMS,
)(_phase_b_body)


_ROWS_BLK = 640
_INVS = float((1.0 + 1e-5) ** -0.5)


def _mlp_kernel(eps_ref, dinv_ref, x_ref, a0_ref, a1_ref, lw_ref, lb_ref,
                w1_ref, b1_ref, g1_ref, be1_ref, w2_ref, b2_ref,
                bng_ref, bnb_ref, out_ref, *, final_bn):
  x = x_ref[...]
  aggr = jnp.concatenate([a0_ref[...], a1_ref[...]], axis=1)
  d = dinv_ref[...]
  sl_msg = jnp.maximum(x + (d * d) * lw_ref[...] + lb_ref[...], 0.0)
  hin = (1.0 + eps_ref[0, 0]) * x + aggr + sl_msg
  t = jnp.dot(hin, w1_ref[...], preferred_element_type=jnp.float32) + b1_ref[...]
  t = jnp.maximum(g1_ref[...] * (t * _INVS) + be1_ref[...], 0.0)
  h = jnp.dot(t, w2_ref[...], preferred_element_type=jnp.float32) + b2_ref[...]
  if final_bn:
    h = jnp.maximum(bng_ref[...] * (h * _INVS) + bnb_ref[...], 0.0)
  out_ref[...] = h


def _mlp_call(final_bn, d_out, eps, dinv, x, a0, a1, lw, lb, w1, b1, g1, be1,
              w2, b2, bng, bnb):
  full = lambda shape: pl.BlockSpec(shape, lambda i: (0, 0))
  grid = NP // _ROWS_BLK
  return pl.pallas_call(
      functools.partial(_mlp_kernel, final_bn=final_bn),
      grid=(grid,),
      in_specs=[
          full((1, 1)),
          pl.BlockSpec((_ROWS_BLK, 1), lambda i: (i, 0)),
          pl.BlockSpec((_ROWS_BLK, D_IN), lambda i: (i, 0)),
          pl.BlockSpec((_ROWS_BLK, 128), lambda i: (i, 0)),
          pl.BlockSpec((_ROWS_BLK, 128), lambda i: (i, 0)),
          full((1, D_IN)), full((1, D_IN)),
          full((D_IN, D_HID)), full((1, D_HID)), full((1, D_HID)),
          full((1, D_HID)),
          full((D_HID, d_out)), full((1, d_out)),
          full((1, d_out)), full((1, d_out)),
      ],
      out_specs=pl.BlockSpec((_ROWS_BLK, d_out), lambda i: (i, 0)),
      out_shape=jax.ShapeDtypeStruct((NP, d_out), jnp.float32),
  )(eps.reshape(1, 1), dinv.reshape(NP, 1), x, a0, a1,
    lw.reshape(1, D_IN), lb.reshape(1, D_IN), w1, b1.reshape(1, D_HID),
    g1.reshape(1, D_HID), be1.reshape(1, D_HID), w2, b2.reshape(1, d_out),
    bng.reshape(1, d_out), bnb.reshape(1, d_out))


def kernel(x, edge_index, edge_weight, eps0, le0_w, le0_b, m0_w1, m0_b1,
           m0_g1, m0_be1, m0_w2, m0_b2, bn0_g, bn0_b, eps1, le1_w, le1_b,
           m1_w1, m1_b1, m1_g1, m1_be1, m1_w2, m1_b2):
  pad = EP - E
  row2 = jnp.concatenate(
      [edge_index[0], jnp.zeros((pad,), jnp.int32)]).reshape(EB, EW)
  col2 = jnp.concatenate(
      [edge_index[1], jnp.full((pad,), N, jnp.int32)]).reshape(EB, EW)
  ew2 = jnp.concatenate(
      [edge_weight, jnp.zeros((pad,), jnp.float32)]).reshape(EB, EW)

  dinv2, norm2 = _phase_a(row2, col2, ew2)
  dinv = dinv2.reshape(NP)

  xp = jnp.pad(x, ((0, NP - N), (0, 0)))
  a0, a1 = _phase_b(x[:, :128], x[:, 128:], row2, col2, norm2,
                    le0_w.reshape(2, 128), le0_b.reshape(2, 128))
  h = _mlp_call(True, D_IN, eps0, dinv, xp, a0, a1, le0_w, le0_b,
                m0_w1, m0_b1, m0_g1, m0_be1, m0_w2, m0_b2, bn0_g, bn0_b)

  b0, b1 = _phase_b(h[:N, :128], h[:N, 128:], row2, col2, norm2,
                    le1_w.reshape(2, 128), le1_b.reshape(2, 128))
  out = _mlp_call(False, D_IN, eps1, dinv, h, b0, b1, le1_w, le1_b,
                  m1_w1, m1_b1, m1_g1, m1_be1, m1_w2, m1_b2,
                  jnp.ones((D_IN,), jnp.float32), jnp.zeros((D_IN,), jnp.float32))
  return out[:N]


# speedup vs baseline: 2.0803x; 1.1055x over previous
import functools

import jax
import jax.numpy as jnp
from jax import lax
from jax.experimental import pallas as pl
from jax.experimental.pallas import tpu as pltpu
from jax.experimental.pallas import tpu_sc as plsc

N = 10000
E = 160000
D_IN = 256
D_HID = 512

NC = 2
NS = 16
L = 16

NP = 10240
EP = 163840
EW = 32
EB = EP // EW
ERT = EB // NS
ERW = EB // (NC * NS)
NDW = NP // NS
DGR = NP // 8
SLAB = 32

EWB = 64
EBB = EP // EWB
ERTB = EBB // NS
SLABB = 16
_RINGB = 2

_SC_MESH = plsc.VectorSubcoreMesh(
    core_axis_name="c", subcore_axis_name="s", num_cores=NC, num_subcores=NS)
_SC_PARAMS = pltpu.CompilerParams(needs_layout_passes=False)

_IOTA = lambda: lax.iota(jnp.int32, L)


def _rsqrt16(v):
  i = lax.bitcast_convert_type(v, jnp.int32)
  i = jnp.full((L,), 0x5F3759DF, dtype=jnp.int32) - lax.shift_right_logical(i, 1)
  y = lax.bitcast_convert_type(i, jnp.float32)
  for _ in range(3):
    y = y * (1.5 - 0.5 * v * y * y)
  return y


def _zero_rows(ref, nrows, ncol16):
  @pl.loop(0, nrows)
  def _(i):
    for k in range(ncol16):
      ref[i, pl.ds(16 * k, 16)] = jnp.zeros((L,), jnp.float32)


def _phase_a_body(row2, col2, ew2, dinv_out, norm_out,
                  rbuf, cbuf, ebuf, c8buf, stage, dchunk2, dbuf, nbuf,
                  degacc, dshared2):
  c = lax.axis_index("c")
  s = lax.axis_index("s")
  zero16 = jnp.zeros((L,), jnp.float32)

  _zero_rows(stage, EW, 8)
  pltpu.sync_copy(stage, degacc.at[pl.ds(s * 80, 32)])
  pltpu.sync_copy(stage, degacc.at[pl.ds(s * 80 + 32, 32)])
  pltpu.sync_copy(stage.at[pl.ds(0, 16)], degacc.at[pl.ds(s * 80 + 64, 16)])
  plsc.subcore_barrier()

  @pl.loop(0, ERT // 8)
  def _(cc):
    base = s * ERT + cc * 8
    pltpu.sync_copy(col2.at[pl.ds(base, 8)], cbuf)
    pltpu.sync_copy(ew2.at[pl.ds(base, 8)], ebuf)

    @pl.loop(0, 8)
    def _(j):
      for i in range(2):
        sl = pl.ds(16 * i, 16)
        cv16 = cbuf[j, sl]
        c8buf[j, sl] = lax.shift_right_logical(cv16, 3)
        ew16 = ebuf[j, sl]
        for q in range(16):
          grp = jnp.bitwise_and(cv16[q], 7)
          ws = jnp.full((L,), ew16[q], jnp.float32)
          for g2 in range(8):
            stage[16 * i + q, pl.ds(16 * g2, 16)] = jnp.where(
                grp == g2, ws, zero16)
      pltpu.sync_copy(stage, degacc.at[c8buf.at[j]], add=True)

  plsc.subcore_barrier()

  @pl.when(s < 10)
  def _():
    for r in range(8):
      pltpu.sync_copy(degacc.at[pl.ds(128 * s + 16 * r, 16)],
                      stage.at[pl.ds(0, 16)])
      for t in range(8):
        tot = zero16
        for rr in range(2):
          for q in range(8):
            tot = jnp.where(_IOTA() == 8 * rr + q,
                            stage[2 * t + rr, pl.ds(16 * q, 16)], tot)
        dchunk2[r, pl.ds(16 * t, 16)] = _rsqrt16(tot + 1.0)

    pltpu.sync_copy(dchunk2, dshared2.at[pl.ds(8 * s, 8)])

    @pl.when(c == 0)
    def _():
      pltpu.sync_copy(dchunk2, dinv_out.at[pl.ds(8 * s, 8)])

  plsc.subcore_barrier()

  pltpu.sync_copy(dshared2, dbuf)

  w = s * NC + c

  @pl.loop(0, ERW // 8)
  def _(cc):
    base = w * ERW + cc * 8
    pltpu.sync_copy(row2.at[pl.ds(base, 8)], rbuf)
    pltpu.sync_copy(col2.at[pl.ds(base, 8)], cbuf)
    pltpu.sync_copy(ew2.at[pl.ds(base, 8)], ebuf)

    @pl.loop(0, 8)
    def _(j):
      for i in range(2):
        sl = pl.ds(16 * i, 16)
        r16 = rbuf[j, sl]
        c16 = cbuf[j, sl]
        dr = plsc.load_gather(
            dbuf, [lax.shift_right_logical(r16, 7),
                   jnp.bitwise_and(r16, 127)])
        dc = plsc.load_gather(
            dbuf, [lax.shift_right_logical(c16, 7),
                   jnp.bitwise_and(c16, 127)])
        nbuf[j, sl] = dr * ebuf[j, sl] * dc

    pltpu.sync_copy(nbuf, norm_out.at[pl.ds(base, 8)])


_phase_a = functools.partial(
    pl.kernel,
    out_type=(jax.ShapeDtypeStruct((NP // 128, 128), jnp.float32),
              jax.ShapeDtypeStruct((EB, EW), jnp.float32)),
    mesh=_SC_MESH,
    scratch_types=[
        pltpu.VMEM((8, EW), jnp.int32),
        pltpu.VMEM((8, EW), jnp.int32),
        pltpu.VMEM((8, EW), jnp.float32),
        pltpu.VMEM((8, EW), jnp.int32),
        pltpu.VMEM((EW, 128), jnp.float32),
        pltpu.VMEM((8, 128), jnp.float32),
        pltpu.VMEM((NP // 128, 128), jnp.float32),
        pltpu.VMEM((8, EW), jnp.float32),
        pltpu.VMEM_SHARED((DGR, 128), jnp.float32),
        pltpu.VMEM_SHARED((NP // 128, 128), jnp.float32),
    ],
    compiler_params=_SC_PARAMS,
)(_phase_a_body)


def _msg_pass_main(xt, row2, col2, norm2, s, rowbuf, colbuf, normbuf,
                   g0, g1, gsem0, gsem1, lwbuf, lbbuf, acc):
  gb = (g0, g1)
  gsems = (gsem0, gsem1)
  lwv = [lwbuf[pl.ds(16 * k, 16)] for k in range(8)]
  lbv = [lbbuf[pl.ds(16 * k, 16)] for k in range(8)]

  def gstart(j, b):
    pltpu.async_copy(xt.at[rowbuf.at[j]], gb[b], gsems[b])

  def gwait(j, b):
    pltpu.make_async_copy(xt.at[rowbuf.at[j]], gb[b], gsems[b]).wait()

  @pl.loop(0, ERTB // SLABB)
  def _(cc):
    base = s * ERTB + cc * SLABB
    pltpu.sync_copy(row2.at[pl.ds(base, SLABB)], rowbuf)
    pltpu.sync_copy(col2.at[pl.ds(base, SLABB)], colbuf)
    pltpu.sync_copy(norm2.at[pl.ds(base, SLABB)], normbuf)
    for b in range(_RINGB):
      gstart(b, b)

    @pl.loop(0, SLABB // _RINGB)
    def _(jj):
      for b in range(_RINGB):
        j = jj * _RINGB + b
        gwait(j, b)
        for eo in range(4):
          nv = normbuf[j, pl.ds(eo * 16, 16)]
          for q in range(16):
            ns = jnp.full((L,), nv[q], jnp.float32)
            e = eo * 16 + q
            for k in range(8):
              sl = pl.ds(16 * k, 16)
              v = gb[b][e, sl] + (ns * lwv[k] + lbv[k])
              gb[b][e, sl] = jnp.maximum(v, 0.0)
        pltpu.sync_copy(gb[b], acc.at[colbuf.at[j]], add=True)

        @pl.when(jj < SLABB // _RINGB - 1)
        def _():
          gstart(j + _RINGB, b)


def _phase_b_body(x0, x1, row2, col2, norm2, lw2, lb2, out0, out1,
                  rowbuf, colbuf, normbuf, g0, g1,
                  gsem0, gsem1, lwbuf, lbbuf, acc):
  c = lax.axis_index("c")
  s = lax.axis_index("s")

  pltpu.sync_copy(lw2.at[c], lwbuf)
  pltpu.sync_copy(lb2.at[c], lbbuf)

  _zero_rows(g0, EWB, 8)
  for t in range(NDW // EWB):
    pltpu.sync_copy(g0, acc.at[pl.ds(s * NDW + EWB * t, EWB)])
  plsc.subcore_barrier()

  args = (row2, col2, norm2, s, rowbuf, colbuf, normbuf, g0, g1,
          gsem0, gsem1, lwbuf, lbbuf, acc)

  @pl.when(c == 0)
  def _():
    _msg_pass_main(x0, *args)

  @pl.when(c == 1)
  def _():
    _msg_pass_main(x1, *args)

  plsc.subcore_barrier()
  for t in range(NDW // EWB):
    sl = pl.ds(s * NDW + EWB * t, EWB)
    pltpu.sync_copy(acc.at[sl], g0)

    @pl.when(c == 0)
    def _():
      pltpu.sync_copy(g0, out0.at[sl])

    @pl.when(c == 1)
    def _():
      pltpu.sync_copy(g0, out1.at[sl])


_phase_b = functools.partial(
    pl.kernel,
    out_type=(jax.ShapeDtypeStruct((NP, 128), jnp.float32),
              jax.ShapeDtypeStruct((NP, 128), jnp.float32)),
    mesh=_SC_MESH,
    scratch_types=[
        pltpu.VMEM((SLABB, EWB), jnp.int32),
        pltpu.VMEM((SLABB, EWB), jnp.int32),
        pltpu.VMEM((SLABB, EWB), jnp.float32),
        pltpu.VMEM((EWB, 128), jnp.float32),
        pltpu.VMEM((EWB, 128), jnp.float32),
        pltpu.SemaphoreType.DMA,
        pltpu.SemaphoreType.DMA,
        pltpu.VMEM((128,), jnp.float32),
        pltpu.VMEM((128,), jnp.float32),
        pltpu.VMEM_SHARED((NP, 128), jnp.float32),
    ],
    compiler_params=_SC_PARAMS,
)(_phase_b_body)


_ROWS_BLK = 640
_INVS = float((1.0 + 1e-5) ** -0.5)


def _mlp_kernel(eps_ref, dinv_ref, x_ref, a0_ref, a1_ref, lw_ref, lb_ref,
                w1_ref, b1_ref, g1_ref, be1_ref, w2_ref, b2_ref,
                bng_ref, bnb_ref, out_ref, *, final_bn):
  x = x_ref[...]
  aggr = jnp.concatenate([a0_ref[...], a1_ref[...]], axis=1)
  d = dinv_ref[...]
  sl_msg = jnp.maximum(x + (d * d) * lw_ref[...] + lb_ref[...], 0.0)
  hin = (1.0 + eps_ref[0, 0]) * x + aggr + sl_msg
  t = jnp.dot(hin, w1_ref[...], preferred_element_type=jnp.float32) + b1_ref[...]
  t = jnp.maximum(g1_ref[...] * (t * _INVS) + be1_ref[...], 0.0)
  h = jnp.dot(t, w2_ref[...], preferred_element_type=jnp.float32) + b2_ref[...]
  if final_bn:
    h = jnp.maximum(bng_ref[...] * (h * _INVS) + bnb_ref[...], 0.0)
  out_ref[...] = h


def _mlp_call(final_bn, d_out, eps, dinv, x, a0, a1, lw, lb, w1, b1, g1, be1,
              w2, b2, bng, bnb):
  full = lambda shape: pl.BlockSpec(shape, lambda i: (0, 0))
  grid = NP // _ROWS_BLK
  return pl.pallas_call(
      functools.partial(_mlp_kernel, final_bn=final_bn),
      grid=(grid,),
      in_specs=[
          full((1, 1)),
          pl.BlockSpec((_ROWS_BLK, 1), lambda i: (i, 0)),
          pl.BlockSpec((_ROWS_BLK, D_IN), lambda i: (i, 0)),
          pl.BlockSpec((_ROWS_BLK, 128), lambda i: (i, 0)),
          pl.BlockSpec((_ROWS_BLK, 128), lambda i: (i, 0)),
          full((1, D_IN)), full((1, D_IN)),
          full((D_IN, D_HID)), full((1, D_HID)), full((1, D_HID)),
          full((1, D_HID)),
          full((D_HID, d_out)), full((1, d_out)),
          full((1, d_out)), full((1, d_out)),
      ],
      out_specs=pl.BlockSpec((_ROWS_BLK, d_out), lambda i: (i, 0)),
      out_shape=jax.ShapeDtypeStruct((NP, d_out), jnp.float32),
  )(eps.reshape(1, 1), dinv.reshape(NP, 1), x, a0, a1,
    lw.reshape(1, D_IN), lb.reshape(1, D_IN), w1, b1.reshape(1, D_HID),
    g1.reshape(1, D_HID), be1.reshape(1, D_HID), w2, b2.reshape(1, d_out),
    bng.reshape(1, d_out), bnb.reshape(1, d_out))


def kernel(x, edge_index, edge_weight, eps0, le0_w, le0_b, m0_w1, m0_b1,
           m0_g1, m0_be1, m0_w2, m0_b2, bn0_g, bn0_b, eps1, le1_w, le1_b,
           m1_w1, m1_b1, m1_g1, m1_be1, m1_w2, m1_b2):
  pad = EP - E
  row2 = jnp.concatenate(
      [edge_index[0], jnp.zeros((pad,), jnp.int32)]).reshape(EB, EW)
  col2 = jnp.concatenate(
      [edge_index[1], jnp.full((pad,), N, jnp.int32)]).reshape(EB, EW)
  ew2 = jnp.concatenate(
      [edge_weight, jnp.zeros((pad,), jnp.float32)]).reshape(EB, EW)

  dinv2, norm2 = _phase_a(row2, col2, ew2)
  dinv = dinv2.reshape(NP)

  row2b = row2.reshape(EBB, EWB)
  col2b = col2.reshape(EBB, EWB)
  norm2b = norm2.reshape(EBB, EWB)

  xp = jnp.pad(x, ((0, NP - N), (0, 0)))
  a0, a1 = _phase_b(x[:, :128], x[:, 128:], row2b, col2b, norm2b,
                    le0_w.reshape(2, 128), le0_b.reshape(2, 128))
  h = _mlp_call(True, D_IN, eps0, dinv, xp, a0, a1, le0_w, le0_b,
                m0_w1, m0_b1, m0_g1, m0_be1, m0_w2, m0_b2, bn0_g, bn0_b)

  b0, b1 = _phase_b(h[:N, :128], h[:N, 128:], row2b, col2b, norm2b,
                    le1_w.reshape(2, 128), le1_b.reshape(2, 128))
  out = _mlp_call(False, D_IN, eps1, dinv, h, b0, b1, le1_w, le1_b,
                  m1_w1, m1_b1, m1_g1, m1_be1, m1_w2, m1_b2,
                  jnp.ones((D_IN,), jnp.float32), jnp.zeros((D_IN,), jnp.float32))
  return out[:N]
